# baseline (speedup 1.0000x reference)
import functools

import jax
import jax.numpy as jnp
from jax import lax
from jax.experimental import pallas as pl
from jax.experimental.pallas import tpu as pltpu
from jax.experimental.pallas import tpu_sc as plsc

N = 10000
D = 256
DH = 128
E = 160000
NT = 16
CH = 128
NCHUNK = 80
GRP = 8
NGRP = NCHUNK // GRP
EPT = CH * NCHUNK
EP = EPT * NT
ACC_ROWS = 10240
PAD_ROWS = ACC_ROWS - N
ROWS_OUT = ACC_ROWS // NT
ZCH = 128
ZITER = ACC_ROWS // NT // ZCH


def _zero_buf(buf, nrow):
    def body(i, _):
        buf[i // 8, pl.ds((i % 8) * 16, 16)] = jnp.zeros((16,), jnp.float32)
        return 0
    lax.fori_loop(0, nrow * 8, body, 0)


def _add_src_offset(sd, coff):
    def body(i, _):
        r = i // 8
        v = i % 8
        sl = pl.ds(v * 16, 16)
        sd[0, r, sl] = sd[0, r, sl] + coff
        return 0
    lax.fori_loop(0, GRP * 8, body, 0)


def _sc_body(xs2, sdp, m2, acc_sh, sd0, sd1, rows0, rows1, sem0, sem1):
    c = lax.axis_index("c")
    s = lax.axis_index("s")
    coff = c * N

    _zero_buf(rows0, CH)
    for b in range(ZITER):
        r0 = (s * ZITER + b) * ZCH
        pltpu.sync_copy(rows0, acc_sh.at[pl.ds(r0, ZCH)])

    plsc.subcore_barrier()

    pltpu.sync_copy(sdp.at[s, 0], sd0)
    _add_src_offset(sd0, coff)
    pltpu.async_copy(xs2.at[sd0.at[0, 0]], rows0, sem0)

    def group_body(g, sd_cur, sd_nxt):
        @pl.when(g + 1 < NGRP)
        def _():
            pltpu.sync_copy(sdp.at[s, g + 1], sd_nxt)
            _add_src_offset(sd_nxt, coff)
        for j in range(GRP):
            if j % 2 == 0:
                r_cur, r_nxt, s_cur, s_nxt = rows0, rows1, sem0, sem1
            else:
                r_cur, r_nxt, s_cur, s_nxt = rows1, rows0, sem1, sem0
            if j < GRP - 1:
                pltpu.async_copy(xs2.at[sd_cur.at[0, j + 1]], r_nxt, s_nxt)
            else:
                @pl.when(g + 1 < NGRP)
                def _():
                    pltpu.async_copy(xs2.at[sd_nxt.at[0, 0]], r_nxt, s_nxt)
            pltpu.make_async_copy(xs2.at[sd_cur.at[0, j]], r_cur, s_cur).wait()
            pltpu.sync_copy(r_cur, acc_sh.at[sd_cur.at[1, j]], add=True)

    def gloop(g, _):
        @pl.when(lax.rem(g, 2) == 0)
        def _():
            group_body(g, sd0, sd1)

        @pl.when(lax.rem(g, 2) == 1)
        def _():
            group_body(g, sd1, sd0)
        return 0
    lax.fori_loop(0, NGRP, gloop, 0)

    plsc.subcore_barrier()

    ob = pl.multiple_of(c * ACC_ROWS + s * ROWS_OUT, 8)
    pltpu.sync_copy(acc_sh.at[pl.ds(s * ROWS_OUT, ROWS_OUT)],
                    m2.at[pl.ds(ob, ROWS_OUT)])


@functools.lru_cache(maxsize=None)
def _make_sc_agg():
    mesh = plsc.VectorSubcoreMesh(core_axis_name="c", subcore_axis_name="s")
    return pl.kernel(
        _sc_body,
        out_type=[jax.ShapeDtypeStruct((2 * ACC_ROWS, DH), jnp.float32)],
        mesh=mesh,
        scratch_types=[
            pltpu.VMEM_SHARED((ACC_ROWS, DH), jnp.float32),
            pltpu.VMEM((2, GRP, CH), jnp.int32),
            pltpu.VMEM((2, GRP, CH), jnp.int32),
            pltpu.VMEM((CH, DH), jnp.float32),
            pltpu.VMEM((CH, DH), jnp.float32),
            pltpu.SemaphoreType.DMA,
            pltpu.SemaphoreType.DMA,
        ],
    )


def _deg_body(sdp, dg2, deg_sh, sd0, sd1, ones128):
    c = lax.axis_index("c")
    s = lax.axis_index("s")

    _zero_buf(ones128, CH)
    for b in range(ZITER):
        r0 = (s * ZITER + b) * ZCH
        pltpu.sync_copy(ones128, deg_sh.at[pl.ds(r0, ZCH)])

    def fill_ones(i, _):
        ones128[i // 8, pl.ds((i % 8) * 16, 16)] = jnp.ones((16,), jnp.float32)
        return 0
    lax.fori_loop(0, CH * 8, fill_ones, 0)

    plsc.subcore_barrier()

    g0 = c * (NGRP // 2)
    pltpu.sync_copy(sdp.at[s, g0], sd0)

    def group_body(g, sd_cur, sd_nxt):
        @pl.when(g + 1 < NGRP // 2)
        def _():
            pltpu.sync_copy(sdp.at[s, g0 + g + 1], sd_nxt)
        for j in range(GRP):
            pltpu.sync_copy(ones128, deg_sh.at[sd_cur.at[1, j]], add=True)

    def gloop(g, _):
        @pl.when(lax.rem(g, 2) == 0)
        def _():
            group_body(g, sd0, sd1)

        @pl.when(lax.rem(g, 2) == 1)
        def _():
            group_body(g, sd1, sd0)
        return 0
    lax.fori_loop(0, NGRP // 2, gloop, 0)

    plsc.subcore_barrier()

    ob = pl.multiple_of(c * ACC_ROWS + s * ROWS_OUT, 8)
    pltpu.sync_copy(deg_sh.at[pl.ds(s * ROWS_OUT, ROWS_OUT)],
                    dg2.at[pl.ds(ob, ROWS_OUT)])


@functools.lru_cache(maxsize=None)
def _make_deg():
    mesh = plsc.VectorSubcoreMesh(core_axis_name="c", subcore_axis_name="s")
    return pl.kernel(
        _deg_body,
        out_type=[jax.ShapeDtypeStruct((2 * ACC_ROWS, DH), jnp.float32)],
        mesh=mesh,
        scratch_types=[
            pltpu.VMEM_SHARED((ACC_ROWS, DH), jnp.float32),
            pltpu.VMEM((2, GRP, CH), jnp.int32),
            pltpu.VMEM((2, GRP, CH), jnp.int32),
            pltpu.VMEM((CH, DH), jnp.float32),
        ],
    )


def _bf(a):
    return a.astype(jnp.bfloat16)


def _tc_bn_body(m2, dg2, hs2, wl, bl, wr, g, beta, out):
    mw = (jnp.dot(_bf(m2[:N]), _bf(wl[:DH, :]), preferred_element_type=jnp.float32)
          + jnp.dot(_bf(m2[ACC_ROWS:ACC_ROWS + N]), _bf(wl[DH:, :]),
                    preferred_element_type=jnp.float32))
    hw = (jnp.dot(_bf(hs2[:N]), _bf(wr[:DH, :]), preferred_element_type=jnp.float32)
          + jnp.dot(_bf(hs2[N:]), _bf(wr[DH:, :]), preferred_element_type=jnp.float32))
    d = jnp.maximum(dg2[:N, :1] + dg2[ACC_ROWS:ACC_ROWS + N, :1], 1.0)
    t = mw / d + bl[...] + hw
    mu = jnp.mean(t, axis=0, keepdims=True)
    var = jnp.mean((t - mu) ** 2, axis=0, keepdims=True)
    h = jnp.maximum((t - mu) * lax.rsqrt(var + 1e-5) * g[...] + beta[...], 0.0)
    out[:N] = h[:, :DH]
    out[N:] = h[:, DH:]


_tc_bn_relu = pl.pallas_call(
    _tc_bn_body,
    out_shape=[jax.ShapeDtypeStruct((2 * N, DH), jnp.float32)],
)

BF = 2000


def _tc_final_body(mlo, mhi, dga, dgb, hlo, hhi, wl, bl, wr, out):
    mw = (jnp.dot(_bf(mlo[...]), _bf(wl[:DH, :]), preferred_element_type=jnp.float32)
          + jnp.dot(_bf(mhi[...]), _bf(wl[DH:, :]), preferred_element_type=jnp.float32))
    hw = (jnp.dot(_bf(hlo[...]), _bf(wr[:DH, :]), preferred_element_type=jnp.float32)
          + jnp.dot(_bf(hhi[...]), _bf(wr[DH:, :]), preferred_element_type=jnp.float32))
    d = jnp.maximum(dga[:, :1] + dgb[:, :1], 1.0)
    t = mw / d + bl[...] + hw
    m = jnp.max(t, axis=1, keepdims=True)
    lse = jnp.log(jnp.sum(jnp.exp(t - m), axis=1, keepdims=True)) + m
    out[...] = t - lse


def _blk(i):
    return (i, 0)


def _rep(i):
    return (0, 0)


_tc_final = pl.pallas_call(
    _tc_final_body,
    grid=(N // BF,),
    in_specs=[pl.BlockSpec((BF, DH), _blk)] * 6
    + [pl.BlockSpec((D, D), _rep), pl.BlockSpec((1, D), _rep),
       pl.BlockSpec((D, D), _rep)],
    out_specs=pl.BlockSpec((BF, D), _blk),
    out_shape=jax.ShapeDtypeStruct((N, D), jnp.float32),
)


def kernel(x, edge_index, Wl0, bl0, Wr0, g0, beta0,
           Wl1, bl1, Wr1, g1, beta1, Wl2, bl2, Wr2):
    src, dst = edge_index[0], edge_index[1]
    ar = jnp.arange(EP - E, dtype=jnp.int32)
    srcp = jnp.concatenate([src, ar % 128])
    dstp = jnp.concatenate([dst, N + (ar % PAD_ROWS)])
    srcc = srcp.reshape(NCHUNK, NT, CH).transpose(1, 0, 2).reshape(NT, NGRP, GRP, CH)
    dstc = dstp.reshape(NCHUNK, NT, CH).transpose(1, 0, 2).reshape(NT, NGRP, GRP, CH)
    sdp = jnp.stack([srcc, dstc], axis=2)
    xs2 = jnp.concatenate([x[:, :DH], x[:, DH:]], axis=0)

    (dg2,) = _make_deg()(sdp)
    (m2,) = _make_sc_agg()(xs2, sdp)
    (hs2,) = _tc_bn_relu(m2, dg2, xs2, Wl0, bl0.reshape(1, D),
                         Wr0, g0.reshape(1, D), beta0.reshape(1, D))
    (m2,) = _make_sc_agg()(hs2, sdp)
    (hs2,) = _tc_bn_relu(m2, dg2, hs2, Wl1, bl1.reshape(1, D),
                         Wr1, g1.reshape(1, D), beta1.reshape(1, D))
    (m2,) = _make_sc_agg()(hs2, sdp)
    out = _tc_final(m2[:N], m2[ACC_ROWS:ACC_ROWS + N],
                    dg2[:N], dg2[ACC_ROWS:ACC_ROWS + N],
                    hs2[:N], hs2[N:], Wl2, bl2.reshape(1, D), Wr2)
    return out


# speedup vs baseline: 1.0078x; 1.0078x over previous
import functools

import jax
import jax.numpy as jnp
from jax import lax
from jax.experimental import pallas as pl
from jax.experimental.pallas import tpu as pltpu
from jax.experimental.pallas import tpu_sc as plsc

N = 10000
D = 256
DH = 128
E = 160000
NT = 16
CH = 128
NCHUNK = 80
GRP = 8
NGRP = NCHUNK // GRP
EPT = CH * NCHUNK
EP = EPT * NT
ACC_ROWS = 10240
PAD_ROWS = ACC_ROWS - N
ROWS_OUT = ACC_ROWS // NT
ZCH = 128
ZITER = ACC_ROWS // NT // ZCH


def _zero_buf(buf, nrow):
    def body(i, _):
        buf[i // 8, pl.ds((i % 8) * 16, 16)] = jnp.zeros((16,), jnp.float32)
        return 0
    lax.fori_loop(0, nrow * 8, body, 0)


def _add_src_offset(sd, coff):
    def body(i, _):
        sl = pl.ds((i % 8) * 16, 16)
        sd[i // 8, sl] = sd[i // 8, sl] + coff
        return 0
    lax.fori_loop(0, GRP * 8, body, 0)


def _sc_body(xs2, sdp, m2, acc_sh, sd0s, sd0d, sd1s, sd1d,
             rows0, rows1, sem0, sem1):
    c = lax.axis_index("c")
    s = lax.axis_index("s")
    coff = c * N

    _zero_buf(rows0, CH)
    for b in range(ZITER):
        r0 = (s * ZITER + b) * ZCH
        pltpu.sync_copy(rows0, acc_sh.at[pl.ds(r0, ZCH)])

    plsc.subcore_barrier()

    def load_group(g, sds, sdd):
        pltpu.sync_copy(sdp.at[0, s, g], sds)
        pltpu.sync_copy(sdp.at[1, s, g], sdd)
        _add_src_offset(sds, coff)

    load_group(0, sd0s, sd0d)
    pltpu.async_copy(xs2.at[sd0s.at[0]], rows0, sem0)

    def group_body(g, sds, sdd, nxs, nxd):
        @pl.when(g + 1 < NGRP)
        def _():
            load_group(g + 1, nxs, nxd)
        for j in range(GRP):
            if j % 2 == 0:
                r_cur, r_nxt, s_cur, s_nxt = rows0, rows1, sem0, sem1
            else:
                r_cur, r_nxt, s_cur, s_nxt = rows1, rows0, sem1, sem0
            if j < GRP - 1:
                pltpu.async_copy(xs2.at[sds.at[j + 1]], r_nxt, s_nxt)
            else:
                @pl.when(g + 1 < NGRP)
                def _():
                    pltpu.async_copy(xs2.at[nxs.at[0]], r_nxt, s_nxt)
            pltpu.make_async_copy(xs2.at[sds.at[j]], r_cur, s_cur).wait()
            pltpu.sync_copy(r_cur, acc_sh.at[sdd.at[j]], add=True)

    def gloop(g, _):
        @pl.when(lax.rem(g, 2) == 0)
        def _():
            group_body(g, sd0s, sd0d, sd1s, sd1d)

        @pl.when(lax.rem(g, 2) == 1)
        def _():
            group_body(g, sd1s, sd1d, sd0s, sd0d)
        return 0
    lax.fori_loop(0, NGRP, gloop, 0)

    plsc.subcore_barrier()

    ob = pl.multiple_of(c * ACC_ROWS + s * ROWS_OUT, 8)
    pltpu.sync_copy(acc_sh.at[pl.ds(s * ROWS_OUT, ROWS_OUT)],
                    m2.at[pl.ds(ob, ROWS_OUT)])


@functools.lru_cache(maxsize=None)
def _make_sc_agg():
    mesh = plsc.VectorSubcoreMesh(core_axis_name="c", subcore_axis_name="s")
    return pl.kernel(
        _sc_body,
        out_type=[jax.ShapeDtypeStruct((2 * ACC_ROWS, DH), jnp.float32)],
        mesh=mesh,
        scratch_types=[
            pltpu.VMEM_SHARED((ACC_ROWS, DH), jnp.float32),
            pltpu.VMEM((GRP, CH), jnp.int32),
            pltpu.VMEM((GRP, CH), jnp.int32),
            pltpu.VMEM((GRP, CH), jnp.int32),
            pltpu.VMEM((GRP, CH), jnp.int32),
            pltpu.VMEM((CH, DH), jnp.float32),
            pltpu.VMEM((CH, DH), jnp.float32),
            pltpu.SemaphoreType.DMA,
            pltpu.SemaphoreType.DMA,
        ],
    )


def _deg_body(sdp, dg2, deg_sh, sd0, sd1, ones128):
    c = lax.axis_index("c")
    s = lax.axis_index("s")

    _zero_buf(ones128, CH)
    for b in range(ZITER):
        r0 = (s * ZITER + b) * ZCH
        pltpu.sync_copy(ones128, deg_sh.at[pl.ds(r0, ZCH)])

    def fill_ones(i, _):
        ones128[i // 8, pl.ds((i % 8) * 16, 16)] = jnp.ones((16,), jnp.float32)
        return 0
    lax.fori_loop(0, CH * 8, fill_ones, 0)

    plsc.subcore_barrier()

    g0 = c * (NGRP // 2)
    pltpu.sync_copy(sdp.at[1, s, g0], sd0)

    def group_body(g, sd_cur, sd_nxt):
        @pl.when(g + 1 < NGRP // 2)
        def _():
            pltpu.sync_copy(sdp.at[1, s, g0 + g + 1], sd_nxt)
        for j in range(GRP):
            pltpu.sync_copy(ones128, deg_sh.at[sd_cur.at[j]], add=True)

    def gloop(g, _):
        @pl.when(lax.rem(g, 2) == 0)
        def _():
            group_body(g, sd0, sd1)

        @pl.when(lax.rem(g, 2) == 1)
        def _():
            group_body(g, sd1, sd0)
        return 0
    lax.fori_loop(0, NGRP // 2, gloop, 0)

    plsc.subcore_barrier()

    ob = pl.multiple_of(c * ACC_ROWS + s * ROWS_OUT, 8)
    pltpu.sync_copy(deg_sh.at[pl.ds(s * ROWS_OUT, ROWS_OUT)],
                    dg2.at[pl.ds(ob, ROWS_OUT)])


@functools.lru_cache(maxsize=None)
def _make_deg():
    mesh = plsc.VectorSubcoreMesh(core_axis_name="c", subcore_axis_name="s")
    return pl.kernel(
        _deg_body,
        out_type=[jax.ShapeDtypeStruct((2 * ACC_ROWS, DH), jnp.float32)],
        mesh=mesh,
        scratch_types=[
            pltpu.VMEM_SHARED((ACC_ROWS, DH), jnp.float32),
            pltpu.VMEM((GRP, CH), jnp.int32),
            pltpu.VMEM((GRP, CH), jnp.int32),
            pltpu.VMEM((CH, DH), jnp.float32),
        ],
    )


def _bf(a):
    return a.astype(jnp.bfloat16)


def _deg_combine_body(dg2, out):
    out[...] = jnp.maximum(dg2[:N] + dg2[ACC_ROWS:ACC_ROWS + N], 1.0)


_deg_combine = pl.pallas_call(
    _deg_combine_body,
    out_shape=jax.ShapeDtypeStruct((N, DH), jnp.float32),
)


def _tc_bn_body(m2, dgc, hs2, wl, bl, wr, g, beta, out):
    mw = (jnp.dot(_bf(m2[:N]), _bf(wl[:DH, :]), preferred_element_type=jnp.float32)
          + jnp.dot(_bf(m2[ACC_ROWS:ACC_ROWS + N]), _bf(wl[DH:, :]),
                    preferred_element_type=jnp.float32))
    hw = (jnp.dot(_bf(hs2[:N]), _bf(wr[:DH, :]), preferred_element_type=jnp.float32)
          + jnp.dot(_bf(hs2[N:]), _bf(wr[DH:, :]), preferred_element_type=jnp.float32))
    t = mw / dgc[:, :1] + bl[...] + hw
    mu = jnp.mean(t, axis=0, keepdims=True)
    var = jnp.mean((t - mu) ** 2, axis=0, keepdims=True)
    h = jnp.maximum((t - mu) * lax.rsqrt(var + 1e-5) * g[...] + beta[...], 0.0)
    out[:N] = h[:, :DH]
    out[N:] = h[:, DH:]


_tc_bn_relu = pl.pallas_call(
    _tc_bn_body,
    out_shape=[jax.ShapeDtypeStruct((2 * N, DH), jnp.float32)],
)

BF = 2000


def _tc_final_body(mlo, mhi, dgc, hlo, hhi, wl, bl, wr, out):
    mw = (jnp.dot(_bf(mlo[...]), _bf(wl[:DH, :]), preferred_element_type=jnp.float32)
          + jnp.dot(_bf(mhi[...]), _bf(wl[DH:, :]), preferred_element_type=jnp.float32))
    hw = (jnp.dot(_bf(hlo[...]), _bf(wr[:DH, :]), preferred_element_type=jnp.float32)
          + jnp.dot(_bf(hhi[...]), _bf(wr[DH:, :]), preferred_element_type=jnp.float32))
    t = mw / dgc[:, :1] + bl[...] + hw
    m = jnp.max(t, axis=1, keepdims=True)
    lse = jnp.log(jnp.sum(jnp.exp(t - m), axis=1, keepdims=True)) + m
    out[...] = t - lse


def _blk(i):
    return (i, 0)


def _rep(i):
    return (0, 0)


_tc_final = pl.pallas_call(
    _tc_final_body,
    grid=(N // BF,),
    in_specs=[pl.BlockSpec((BF, DH), _blk)] * 5
    + [pl.BlockSpec((D, D), _rep), pl.BlockSpec((1, D), _rep),
       pl.BlockSpec((D, D), _rep)],
    out_specs=pl.BlockSpec((BF, D), _blk),
    out_shape=jax.ShapeDtypeStruct((N, D), jnp.float32),
)


def kernel(x, edge_index, Wl0, bl0, Wr0, g0, beta0,
           Wl1, bl1, Wr1, g1, beta1, Wl2, bl2, Wr2):
    src, dst = edge_index[0], edge_index[1]
    ar = jnp.arange(EP - E, dtype=jnp.int32)
    srcp = jnp.concatenate([src, ar % 128])
    dstp = jnp.concatenate([dst, N + (ar % PAD_ROWS)])
    sdp = jnp.stack([srcp, dstp]).reshape(2, NT, NGRP, GRP, CH)
    xs2 = jnp.concatenate([x[:, :DH], x[:, DH:]], axis=0)

    (dg2,) = _make_deg()(sdp)
    (m2,) = _make_sc_agg()(xs2, sdp)
    dgc = _deg_combine(dg2)
    (hs2,) = _tc_bn_relu(m2, dgc, xs2, Wl0, bl0.reshape(1, D),
                         Wr0, g0.reshape(1, D), beta0.reshape(1, D))
    (m2,) = _make_sc_agg()(hs2, sdp)
    (hs2,) = _tc_bn_relu(m2, dgc, hs2, Wl1, bl1.reshape(1, D),
                         Wr1, g1.reshape(1, D), beta1.reshape(1, D))
    (m2,) = _make_sc_agg()(hs2, sdp)
    out = _tc_final(m2[:N], m2[ACC_ROWS:ACC_ROWS + N], dgc,
                    hs2[:N], hs2[N:], Wl2, bl2.reshape(1, D), Wr2)
    return out


# speedup vs baseline: 1.0188x; 1.0110x over previous
import functools

import jax
import jax.numpy as jnp
from jax import lax
from jax.experimental import pallas as pl
from jax.experimental.pallas import tpu as pltpu
from jax.experimental.pallas import tpu_sc as plsc

N = 10000
D = 256
DH = 128
E = 160000
NT = 16
CH = 128
NCHUNK = 80
GRP = 8
NGRP = NCHUNK // GRP
EPT = CH * NCHUNK
EP = EPT * NT
ACC_ROWS = 10240
PAD_ROWS = ACC_ROWS - N
ROWS_OUT = ACC_ROWS // NT
ZCH = 128
ZITER = ACC_ROWS // NT // ZCH


def _zero_buf(buf, nrow):
    def body(i, _):
        buf[i // 8, pl.ds((i % 8) * 16, 16)] = jnp.zeros((16,), jnp.float32)
        return 0
    lax.fori_loop(0, nrow * 8, body, 0)


def _add_src_offset(sd, coff):
    def body(i, _):
        sl = pl.ds((i % 8) * 16, 16)
        sd[i // 8, sl] = sd[i // 8, sl] + coff
        return 0
    lax.fori_loop(0, GRP * 8, body, 0)


def _sc_body(xs2, sdp, m2, acc_sh, sd0s, sd0d, sd1s, sd1d,
             rows0, rows1, sem0, sem1):
    c = lax.axis_index("c")
    s = lax.axis_index("s")
    coff = c * N

    _zero_buf(rows0, CH)
    for b in range(ZITER):
        r0 = (s * ZITER + b) * ZCH
        pltpu.sync_copy(rows0, acc_sh.at[pl.ds(r0, ZCH)])

    plsc.subcore_barrier()

    def load_group(g, sds, sdd):
        pltpu.sync_copy(sdp.at[0, s, g], sds)
        pltpu.sync_copy(sdp.at[1, s, g], sdd)
        _add_src_offset(sds, coff)

    load_group(0, sd0s, sd0d)
    pltpu.async_copy(xs2.at[sd0s.at[0]], rows0, sem0)

    def group_body(g, sds, sdd, nxs, nxd):
        @pl.when(g + 1 < NGRP)
        def _():
            load_group(g + 1, nxs, nxd)
        for j in range(GRP):
            if j % 2 == 0:
                r_cur, r_nxt, s_cur, s_nxt = rows0, rows1, sem0, sem1
            else:
                r_cur, r_nxt, s_cur, s_nxt = rows1, rows0, sem1, sem0
            if j < GRP - 1:
                pltpu.async_copy(xs2.at[sds.at[j + 1]], r_nxt, s_nxt)
            else:
                @pl.when(g + 1 < NGRP)
                def _():
                    pltpu.async_copy(xs2.at[nxs.at[0]], r_nxt, s_nxt)
            pltpu.make_async_copy(xs2.at[sds.at[j]], r_cur, s_cur).wait()
            pltpu.sync_copy(r_cur, acc_sh.at[sdd.at[j]], add=True)

    def gloop(g, _):
        @pl.when(lax.rem(g, 2) == 0)
        def _():
            group_body(g, sd0s, sd0d, sd1s, sd1d)

        @pl.when(lax.rem(g, 2) == 1)
        def _():
            group_body(g, sd1s, sd1d, sd0s, sd0d)
        return 0
    lax.fori_loop(0, NGRP, gloop, 0)

    plsc.subcore_barrier()

    ob = pl.multiple_of(c * ACC_ROWS + s * ROWS_OUT, 8)
    pltpu.sync_copy(acc_sh.at[pl.ds(s * ROWS_OUT, ROWS_OUT)],
                    m2.at[pl.ds(ob, ROWS_OUT)])


@functools.lru_cache(maxsize=None)
def _make_sc_agg():
    mesh = plsc.VectorSubcoreMesh(core_axis_name="c", subcore_axis_name="s")
    return pl.kernel(
        _sc_body,
        out_type=[jax.ShapeDtypeStruct((2 * ACC_ROWS, DH), jnp.float32)],
        mesh=mesh,
        scratch_types=[
            pltpu.VMEM_SHARED((ACC_ROWS, DH), jnp.float32),
            pltpu.VMEM((GRP, CH), jnp.int32),
            pltpu.VMEM((GRP, CH), jnp.int32),
            pltpu.VMEM((GRP, CH), jnp.int32),
            pltpu.VMEM((GRP, CH), jnp.int32),
            pltpu.VMEM((CH, DH), jnp.float32),
            pltpu.VMEM((CH, DH), jnp.float32),
            pltpu.SemaphoreType.DMA,
            pltpu.SemaphoreType.DMA,
        ],
    )


def _deg_body(sdp, dg2, deg_sh, sd0, sd1, ones128):
    c = lax.axis_index("c")
    s = lax.axis_index("s")

    _zero_buf(ones128, CH)
    for b in range(ZITER):
        r0 = (s * ZITER + b) * ZCH
        pltpu.sync_copy(ones128, deg_sh.at[pl.ds(r0, ZCH)])

    def fill_ones(i, _):
        ones128[i // 8, pl.ds((i % 8) * 16, 16)] = jnp.ones((16,), jnp.float32)
        return 0
    lax.fori_loop(0, CH * 8, fill_ones, 0)

    plsc.subcore_barrier()

    g0 = c * (NGRP // 2)
    pltpu.sync_copy(sdp.at[1, s, g0], sd0)

    def group_body(g, sd_cur, sd_nxt):
        @pl.when(g + 1 < NGRP // 2)
        def _():
            pltpu.sync_copy(sdp.at[1, s, g0 + g + 1], sd_nxt)
        for j in range(GRP):
            pltpu.sync_copy(ones128, deg_sh.at[sd_cur.at[j]], add=True)

    def gloop(g, _):
        @pl.when(lax.rem(g, 2) == 0)
        def _():
            group_body(g, sd0, sd1)

        @pl.when(lax.rem(g, 2) == 1)
        def _():
            group_body(g, sd1, sd0)
        return 0
    lax.fori_loop(0, NGRP // 2, gloop, 0)

    plsc.subcore_barrier()

    ob = pl.multiple_of(c * ACC_ROWS + s * ROWS_OUT, 8)
    pltpu.sync_copy(deg_sh.at[pl.ds(s * ROWS_OUT, ROWS_OUT)],
                    dg2.at[pl.ds(ob, ROWS_OUT)])


@functools.lru_cache(maxsize=None)
def _make_deg():
    mesh = plsc.VectorSubcoreMesh(core_axis_name="c", subcore_axis_name="s")
    return pl.kernel(
        _deg_body,
        out_type=[jax.ShapeDtypeStruct((2 * ACC_ROWS, DH), jnp.float32)],
        mesh=mesh,
        scratch_types=[
            pltpu.VMEM_SHARED((ACC_ROWS, DH), jnp.float32),
            pltpu.VMEM((GRP, CH), jnp.int32),
            pltpu.VMEM((GRP, CH), jnp.int32),
            pltpu.VMEM((CH, DH), jnp.float32),
        ],
    )


def _bf(a):
    return a.astype(jnp.bfloat16)


def _deg_combine_body(dg2, out):
    out[...] = jnp.maximum(dg2[:N] + dg2[ACC_ROWS:ACC_ROWS + N], 1.0)


_deg_combine = pl.pallas_call(
    _deg_combine_body,
    out_shape=jax.ShapeDtypeStruct((N, DH), jnp.float32),
)


def _tc_bn_body(m2, dgc, hs2, wl, bl, wr, g, beta, out):
    mw = (jnp.dot(_bf(m2[:N]), _bf(wl[:DH, :]), preferred_element_type=jnp.float32)
          + jnp.dot(_bf(m2[ACC_ROWS:ACC_ROWS + N]), _bf(wl[DH:, :]),
                    preferred_element_type=jnp.float32))
    hw = (jnp.dot(_bf(hs2[:N]), _bf(wr[:DH, :]), preferred_element_type=jnp.float32)
          + jnp.dot(_bf(hs2[N:]), _bf(wr[DH:, :]), preferred_element_type=jnp.float32))
    t = mw / dgc[:, :1] + bl[...] + hw
    mu = jnp.mean(t, axis=0, keepdims=True)
    var = jnp.mean((t - mu) ** 2, axis=0, keepdims=True)
    h = jnp.maximum((t - mu) * lax.rsqrt(var + 1e-5) * g[...] + beta[...], 0.0)
    out[:N] = h[:, :DH]
    out[N:] = h[:, DH:]


_tc_bn_relu = pl.pallas_call(
    _tc_bn_body,
    out_shape=[jax.ShapeDtypeStruct((2 * N, DH), jnp.float32)],
)

BF = 2000


def _tc_final_body(m2, dgc, hlo, hhi, wl, bl, wr, out):
    i = pl.program_id(0)
    mlo = m2[pl.ds(i * BF, BF)]
    mhi = m2[pl.ds(ACC_ROWS + i * BF, BF)]
    mw = (jnp.dot(_bf(mlo), _bf(wl[:DH, :]), preferred_element_type=jnp.float32)
          + jnp.dot(_bf(mhi), _bf(wl[DH:, :]), preferred_element_type=jnp.float32))
    hw = (jnp.dot(_bf(hlo[...]), _bf(wr[:DH, :]), preferred_element_type=jnp.float32)
          + jnp.dot(_bf(hhi[...]), _bf(wr[DH:, :]), preferred_element_type=jnp.float32))
    t = mw / dgc[:, :1] + bl[...] + hw
    m = jnp.max(t, axis=1, keepdims=True)
    lse = jnp.log(jnp.sum(jnp.exp(t - m), axis=1, keepdims=True)) + m
    out[...] = t - lse


def _blk(i):
    return (i, 0)


def _rep(i):
    return (0, 0)


_tc_final = pl.pallas_call(
    _tc_final_body,
    grid=(N // BF,),
    in_specs=[pl.BlockSpec((2 * ACC_ROWS, DH), _rep)]
    + [pl.BlockSpec((BF, DH), _blk)] * 3
    + [pl.BlockSpec((D, D), _rep), pl.BlockSpec((1, D), _rep),
       pl.BlockSpec((D, D), _rep)],
    out_specs=pl.BlockSpec((BF, D), _blk),
    out_shape=jax.ShapeDtypeStruct((N, D), jnp.float32),
)


def kernel(x, edge_index, Wl0, bl0, Wr0, g0, beta0,
           Wl1, bl1, Wr1, g1, beta1, Wl2, bl2, Wr2):
    src, dst = edge_index[0], edge_index[1]
    ar = jnp.arange(EP - E, dtype=jnp.int32)
    srcp = jnp.concatenate([src, ar % 128])
    dstp = jnp.concatenate([dst, N + (ar % PAD_ROWS)])
    sdp = (jnp.stack([srcp, dstp])
           .reshape(2, NCHUNK, NT, CH)
           .transpose(0, 2, 1, 3)
           .reshape(2, NT, NGRP, GRP, CH))
    xs2 = jnp.concatenate([x[:, :DH], x[:, DH:]], axis=0)

    (dg2,) = _make_deg()(sdp)
    (m2,) = _make_sc_agg()(xs2, sdp)
    dgc = _deg_combine(dg2)
    (hs2,) = _tc_bn_relu(m2, dgc, xs2, Wl0, bl0.reshape(1, D),
                         Wr0, g0.reshape(1, D), beta0.reshape(1, D))
    (m2,) = _make_sc_agg()(hs2, sdp)
    (hs2,) = _tc_bn_relu(m2, dgc, hs2, Wl1, bl1.reshape(1, D),
                         Wr1, g1.reshape(1, D), beta1.reshape(1, D))
    (m2,) = _make_sc_agg()(hs2, sdp)
    out = _tc_final(m2, dgc, hs2[:N], hs2[N:], Wl2, bl2.reshape(1, D), Wr2)
    return out
